# baseline (speedup 1.0000x reference)
import functools

import jax
import jax.numpy as jnp
from jax import lax
from jax.experimental import pallas as pl
from jax.experimental.pallas import tpu as pltpu
from jax.experimental.pallas import tpu_sc as plsc

BATCH = 16384
D = 64
V = 1000000
NW = 32
SLICE = 31232
VC = 512
NWIN = SLICE // VC
ROWS_PAD = BATCH + 16
IDXC = 2048
NIDXC = BATCH // IDXC
OCAP = 1024


def _iota16():
    return lax.iota(jnp.int32, 16)


def _extract_body(user_h, pos_h, neg_h, utt_h, itt_h,
                  urows_h, prows_h, nrows_h,
                  idxbuf, oeid_u, ovl_u, oeid_p, ovl_p, oeid_n, ovl_n,
                  ubuf, ibuf, tu, ti, meid, moff, srows, sem):
    wid = lax.axis_index("s") * 2 + lax.axis_index("c")
    w0 = wid * SLICE
    is_last = wid == NW - 1
    w_end = jnp.where(is_last, V, w0 + SLICE)
    zero16 = jnp.zeros((16,), jnp.int32)

    def scan_stream(src_h, oeid, ovl):
        def chunk(c, cnt):
            pltpu.sync_copy(src_h.at[pl.ds(c * IDXC, IDXC)], idxbuf)

            def blk(b, cnt):
                off = pl.multiple_of(b * 16, 16)
                v = idxbuf[pl.ds(off, 16)]
                m = (v >= w0) & (v < w_end)
                mi = m.astype(jnp.int32)
                pos = cnt + plsc.cumsum(mi) - mi
                eid = c * IDXC + b * 16 + _iota16()
                plsc.store_scatter(oeid, [pos], eid, mask=m)
                plsc.store_scatter(ovl, [pos], v - w0, mask=m)
                return cnt + plsc.all_reduce_population_count(m)

            return lax.fori_loop(0, IDXC // 16, blk, cnt)

        cnt = lax.fori_loop(0, NIDXC, chunk, zero16)
        return jnp.max(cnt)

    no_u = scan_stream(user_h, oeid_u, ovl_u)
    no_p = scan_stream(pos_h, oeid_p, ovl_p)
    no_n = scan_stream(neg_h, oeid_n, ovl_n)

    def extract(buf, k, oeid, ovl, no, dst_h):
        for q in range(4):
            meid[pl.ds(q * 16, 16)] = jnp.full((16,), ROWS_PAD - 16, jnp.int32)
            moff[pl.ds(q * 16, 16)] = jnp.zeros((16,), jnp.int32)

        def blk(b, mc):
            off = pl.multiple_of(b * 16, 16)
            ov = ovl[pl.ds(off, 16)]
            oe = oeid[pl.ds(off, 16)]
            valid = (b * 16 + _iota16()) < no
            m = valid & ((ov >> 9) == k)
            mi = m.astype(jnp.int32)
            pos = mc + plsc.cumsum(mi) - mi
            plsc.store_scatter(meid, [pos], oe, mask=m)
            plsc.store_scatter(moff, [pos], ov & 511, mask=m)
            return mc + plsc.all_reduce_population_count(m)

        nblk = (no + 15) >> 4
        mc = lax.fori_loop(0, nblk, blk, zero16)
        nm = jnp.max(mc)

        for b in range(4):
            @pl.when(b * 16 < nm)
            def _(b=b):
                offs = moff[pl.ds(b * 16, 16)]
                jr = b * 16 + _iota16()
                for d in range(D):
                    ds = jnp.full((16,), d, jnp.int32)
                    vals = plsc.load_gather(buf, [ds, offs])
                    plsc.store_scatter(srows, [jr, ds], vals)
                ids = plsc.load_gather(meid, [jr])
                pltpu.async_copy(
                    srows.at[pl.ds(b * 16, 16), :], dst_h.at[ids], sem
                ).wait()

    def window(k, carry):
        sk = pl.multiple_of(w0 + k * VC, 128)
        pltpu.sync_copy(utt_h.at[:, pl.ds(sk, VC)], ubuf)
        pltpu.sync_copy(itt_h.at[:, pl.ds(sk, VC)], ibuf)
        extract(ubuf, k, oeid_u, ovl_u, no_u, urows_h)
        extract(ibuf, k, oeid_p, ovl_p, no_p, prows_h)
        extract(ibuf, k, oeid_n, ovl_n, no_n, nrows_h)
        return carry

    kc = jnp.where(is_last, NWIN + 1, NWIN)
    lax.fori_loop(0, kc, window, 0)

    @pl.when(is_last)
    def _():
        pltpu.sync_copy(utt_h.at[:, pl.ds(V - 64, 64)], tu)
        pltpu.sync_copy(itt_h.at[:, pl.ds(V - 64, 64)], ti)
        for (oeid, ovl, no, dst_h, buf) in (
            (oeid_u, ovl_u, no_u, urows_h, tu),
            (oeid_p, ovl_p, no_p, prows_h, ti),
            (oeid_n, ovl_n, no_n, nrows_h, ti),
        ):
            def blk(b, mc, oeid=oeid, ovl=ovl, no=no):
                off = pl.multiple_of(b * 16, 16)
                ov = ovl[pl.ds(off, 16)]
                oe = oeid[pl.ds(off, 16)]
                valid = (b * 16 + _iota16()) < no
                m = valid & ((ov >> 9) == NWIN + 1)
                mi = m.astype(jnp.int32)
                pos = mc + plsc.cumsum(mi) - mi
                plsc.store_scatter(meid, [pos], oe, mask=m)
                plsc.store_scatter(moff, [pos], ov - (NWIN + 1) * VC, mask=m)
                return mc + plsc.all_reduce_population_count(m)

            for q in range(4):
                meid[pl.ds(q * 16, 16)] = jnp.full((16,), ROWS_PAD - 16, jnp.int32)
                moff[pl.ds(q * 16, 16)] = jnp.zeros((16,), jnp.int32)
            mc = lax.fori_loop(0, (no + 15) >> 4, blk, zero16)
            nm = jnp.max(mc)

            for b in range(4):
                @pl.when(b * 16 < nm)
                def _(b=b, dst_h=dst_h, buf=buf):
                    offs = moff[pl.ds(b * 16, 16)]
                    jr = b * 16 + _iota16()
                    for d in range(D):
                        ds = jnp.full((16,), d, jnp.int32)
                        vals = plsc.load_gather(buf, [ds, offs])
                        plsc.store_scatter(srows, [jr, ds], vals)
                    ids = plsc.load_gather(meid, [jr])
                    pltpu.async_copy(
                        srows.at[pl.ds(b * 16, 16), :], dst_h.at[ids], sem
                    ).wait()


def _dot_body(urows_h, prows_h, nrows_h, outp_h, outn_h,
              ub, pb, nb, outp_v, outn_v, sem):
    wid = lax.axis_index("s") * 2 + lax.axis_index("c")
    base = wid * (BATCH // NW)

    def chunk(c, carry):
        r0 = pl.multiple_of(base + c * 128, 8)
        pltpu.sync_copy(urows_h.at[pl.ds(r0, 128), :], ub)
        pltpu.sync_copy(prows_h.at[pl.ds(r0, 128), :], pb)
        pltpu.sync_copy(nrows_h.at[pl.ds(r0, 128), :], nb)

        def group(g, carry):
            rows = g * 16 + _iota16()

            def col(d, accs):
                ap, an = accs
                cols = jnp.full((16,), d, jnp.int32)
                u = plsc.load_gather(ub, [rows, cols])
                p = plsc.load_gather(pb, [rows, cols])
                n = plsc.load_gather(nb, [rows, cols])
                return ap + u * p, an + u * n

            zero = jnp.zeros((16,), jnp.float32)
            ap, an = lax.fori_loop(0, D, col, (zero, zero))
            o = c * 128 + g * 16
            outp_v[pl.ds(o, 16)] = ap
            outn_v[pl.ds(o, 16)] = an
            return carry

        lax.fori_loop(0, 8, group, 0)
        return carry

    lax.fori_loop(0, 4, chunk, 0)
    pltpu.sync_copy(outp_v, outp_h.at[pl.ds(base, BATCH // NW)])
    pltpu.sync_copy(outn_v, outn_h.at[pl.ds(base, BATCH // NW)])


@jax.jit
def kernel(user, pos_item, neg_item, user_table, item_table):
    mesh = plsc.VectorSubcoreMesh(core_axis_name="c", subcore_axis_name="s")
    params = pltpu.CompilerParams(needs_layout_passes=False)
    extract = pl.kernel(
        _extract_body,
        mesh=mesh,
        compiler_params=params,
        out_type=(
            jax.ShapeDtypeStruct((ROWS_PAD, 128), jnp.float32),
            jax.ShapeDtypeStruct((ROWS_PAD, 128), jnp.float32),
            jax.ShapeDtypeStruct((ROWS_PAD, 128), jnp.float32),
        ),
        scratch_types=[
            pltpu.VMEM((IDXC,), jnp.int32),
            pltpu.VMEM((OCAP,), jnp.int32),
            pltpu.VMEM((OCAP,), jnp.int32),
            pltpu.VMEM((OCAP,), jnp.int32),
            pltpu.VMEM((OCAP,), jnp.int32),
            pltpu.VMEM((OCAP,), jnp.int32),
            pltpu.VMEM((OCAP,), jnp.int32),
            pltpu.VMEM((D, VC), jnp.float32),
            pltpu.VMEM((D, VC), jnp.float32),
            pltpu.VMEM((D, 64), jnp.float32),
            pltpu.VMEM((D, 64), jnp.float32),
            pltpu.VMEM((64,), jnp.int32),
            pltpu.VMEM((64,), jnp.int32),
            pltpu.VMEM((64, 128), jnp.float32),
            pltpu.SemaphoreType.DMA,
        ],
    )
    dot = pl.kernel(
        _dot_body,
        mesh=mesh,
        compiler_params=params,
        out_type=(
            jax.ShapeDtypeStruct((BATCH,), jnp.float32),
            jax.ShapeDtypeStruct((BATCH,), jnp.float32),
        ),
        scratch_types=[
            pltpu.VMEM((128, 128), jnp.float32),
            pltpu.VMEM((128, 128), jnp.float32),
            pltpu.VMEM((128, 128), jnp.float32),
            pltpu.VMEM((BATCH // NW,), jnp.float32),
            pltpu.VMEM((BATCH // NW,), jnp.float32),
            pltpu.SemaphoreType.DMA,
        ],
    )
    urows, prows, nrows = extract(
        user, pos_item, neg_item, user_table.T, item_table.T
    )
    return dot(urows, prows, nrows)
